# baseline (speedup 1.0000x reference)
import functools

import jax
import jax.numpy as jnp
from jax import lax
from jax.experimental import pallas as pl
from jax.experimental.pallas import tpu as pltpu
from jax.experimental.pallas import tpu_sc as plsc

_SEQ_STEP = 16
_NSLOT = 2


@functools.lru_cache(maxsize=None)
def _make_gather(b0, b1, dim):
    info = plsc.get_sparse_core_info()
    nc, ns = info.num_cores, info.num_subcores
    nw = nc * ns
    seq_per_w = b0 // nw
    n_steps = seq_per_w // _SEQ_STEP
    step_rows = _SEQ_STEP * b1
    b_per_w = seq_per_w * b1
    mesh = plsc.VectorSubcoreMesh(core_axis_name="c", subcore_axis_name="s")

    @functools.partial(
        pl.kernel,
        mesh=mesh,
        out_type=jax.ShapeDtypeStruct((b0, b1, dim), jnp.float32),
        scratch_types=[
            pltpu.VMEM((seq_per_w, b1), jnp.int32),
            pltpu.VMEM((_NSLOT, _SEQ_STEP, b1, dim), jnp.float32),
            pltpu.SemaphoreType.DMA((_NSLOT,)),
            pltpu.SemaphoreType.DMA((_NSLOT,)),
            pltpu.SemaphoreType.DMA,
        ],
        compiler_params=pltpu.CompilerParams(use_tc_tiling_on_sc=False),
    )
    def gather_kernel(table_hbm, idx_hbm, out_hbm, idx_v, rows_v, gsem, osem, isem):
        wid = lax.axis_index("s") * nc + lax.axis_index("c")
        seq0 = wid * seq_per_w
        pltpu.async_copy(idx_hbm.at[pl.ds(seq0, seq_per_w)], idx_v, isem).wait()

        def seq_gather(s, j):
            b = s % _NSLOT
            return pltpu.make_async_copy(
                table_hbm.at[idx_v.at[s * _SEQ_STEP + j]],
                rows_v.at[b, j],
                gsem.at[b],
            )

        def gather_start(s):
            for j in range(_SEQ_STEP):
                seq_gather(s, j).start()

        def gather_wait(s):
            for j in range(_SEQ_STEP):
                seq_gather(s, j).wait()

        def out_copy(s):
            b = s % _NSLOT
            return pltpu.make_async_copy(
                rows_v.at[b],
                out_hbm.at[pl.ds(seq0 + s * _SEQ_STEP, _SEQ_STEP)],
                osem.at[b],
            )

        for s in range(n_steps):
            if s >= _NSLOT:
                out_copy(s - _NSLOT).wait()
            gather_start(s)
            if s >= 1:
                gather_wait(s - 1)
                out_copy(s - 1).start()
        gather_wait(n_steps - 1)
        out_copy(n_steps - 1).start()
        for s in range(max(n_steps - _NSLOT, 0), n_steps):
            out_copy(s).wait()

    return gather_kernel


def kernel(token_ids, embedding_matrix):
    b0, b1 = token_ids.shape
    _, d = embedding_matrix.shape
    return _make_gather(b0, b1, d)(embedding_matrix, token_ids.astype(jnp.int32))
